# speedup vs baseline: 2.9601x; 2.9601x over previous
import jax
import jax.numpy as jnp
from jax import lax
from jax.experimental import pallas as pl
from jax.experimental.pallas import tpu as pltpu
from jax.experimental.pallas import tpu_sc as plsc

NUM_CORES = 2
NUM_SUBCORES = 16
NUM_WORKERS = NUM_CORES * NUM_SUBCORES
CHUNK = 128
EMB = 128


def _gather_body(idx_hbm, table_hbm, out_hbm, idx_v, rows_v, sem):
    wid = lax.axis_index("s") * NUM_CORES + lax.axis_index("c")
    k = idx_hbm.shape[1]
    base = wid * (k * CHUNK)
    pltpu.sync_copy(idx_hbm.at[wid], idx_v)

    def step(j, carry):
        pltpu.async_copy(table_hbm.at[idx_v.at[j]], rows_v, sem).wait()
        pltpu.sync_copy(rows_v, out_hbm.at[pl.ds(base + j * CHUNK, CHUNK)])
        return carry

    lax.fori_loop(0, k, step, 0)


@jax.jit
def kernel(token_ids, weight):
    b, s = token_ids.shape
    total = b * s
    k = total // (NUM_WORKERS * CHUNK)
    idx = token_ids.astype(jnp.int32).reshape(NUM_WORKERS, k, CHUNK)
    mesh = plsc.VectorSubcoreMesh(core_axis_name="c", subcore_axis_name="s")
    out = pl.kernel(
        _gather_body,
        out_type=jax.ShapeDtypeStruct((total, EMB), jnp.float32),
        mesh=mesh,
        scratch_types=[
            pltpu.VMEM((k, CHUNK), jnp.int32),
            pltpu.VMEM((CHUNK, EMB), jnp.float32),
            pltpu.SemaphoreType.DMA,
        ],
    )(idx, weight)
    return out.reshape(b, s, EMB)


# speedup vs baseline: 3.3325x; 1.1258x over previous
import jax
import jax.numpy as jnp
from jax import lax
from jax.experimental import pallas as pl
from jax.experimental.pallas import tpu as pltpu
from jax.experimental.pallas import tpu_sc as plsc

NUM_CORES = 2
NUM_SUBCORES = 16
NUM_WORKERS = NUM_CORES * NUM_SUBCORES
CHUNK = 128
EMB = 128


NBUF = 4


def _gather_body(idx_hbm, table_hbm, out_hbm, idx_v, rows_v, gsem, wsem):
    wid = lax.axis_index("s") * NUM_CORES + lax.axis_index("c")
    k = idx_hbm.shape[1]
    base = wid * (k * CHUNK)
    pltpu.sync_copy(idx_hbm.at[wid], idx_v)

    def issue_gather(g, buf):
        return pltpu.async_copy(
            table_hbm.at[idx_v.at[g]], rows_v.at[buf], gsem.at[buf]
        )

    def issue_writeback(g, buf):
        return pltpu.async_copy(
            rows_v.at[buf],
            out_hbm.at[pl.ds(base + g * CHUNK, CHUNK)],
            wsem.at[buf],
        )

    gd = [None] * NBUF
    wd = [None] * NBUF
    for g in range(min(NBUF, k)):
        gd[g % NBUF] = issue_gather(g, g % NBUF)
    for g in range(k):
        buf = g % NBUF
        gd[buf].wait()
        wd[buf] = issue_writeback(g, buf)
        nxt = g + NBUF
        if nxt < k:
            wd[buf].wait()
            gd[buf] = issue_gather(nxt, buf)
    for g in range(max(k - NBUF, 0), k):
        wd[g % NBUF].wait()


@jax.jit
def kernel(token_ids, weight):
    b, s = token_ids.shape
    total = b * s
    k = total // (NUM_WORKERS * CHUNK)
    idx = token_ids.astype(jnp.int32).reshape(NUM_WORKERS, k, CHUNK)
    mesh = plsc.VectorSubcoreMesh(core_axis_name="c", subcore_axis_name="s")
    out = pl.kernel(
        _gather_body,
        out_type=jax.ShapeDtypeStruct((total, EMB), jnp.float32),
        mesh=mesh,
        scratch_types=[
            pltpu.VMEM((k, CHUNK), jnp.int32),
            pltpu.VMEM((NBUF, CHUNK, EMB), jnp.float32),
            pltpu.SemaphoreType.DMA((NBUF,)),
            pltpu.SemaphoreType.DMA((NBUF,)),
        ],
    )(idx, weight)
    return out.reshape(b, s, EMB)
